# initial kernel scaffold (unmeasured)
import jax
import jax.numpy as jnp
from jax import lax
from jax.experimental import pallas as pl
from jax.experimental.pallas import tpu as pltpu

N_DEV = 8
SQ = 1024
SKV = 8192
KV_PER = 1024
H_PER = 8
DH = 128
QBLK = 128
SCALE = 0.08838834764831843


def kernel(x, Wq, K_ext, V_ext, Wo):
    xb = x.astype(jnp.bfloat16)
    wqb = Wq.astype(jnp.bfloat16)
    wob = Wo.astype(jnp.bfloat16)
    kb = K_ext.astype(jnp.bfloat16)
    vb = V_ext.astype(jnp.bfloat16)

    def body(x_ref, wq_ref, k_hbm, v_hbm, wo_ref, out_ref,
             kg, vg, q_s, ctx_s, p_s,
             ksend, krecv, vsend, vrecv, credit):
        me = lax.axis_index("i")

        barrier = pltpu.get_barrier_semaphore()
        for off in range(1, N_DEV):
            peer = lax.rem(me + off, N_DEV)
            pl.semaphore_signal(
                barrier, inc=1,
                device_id=(peer,), device_id_type=pl.DeviceIdType.MESH,
            )
        pl.semaphore_wait(barrier, N_DEV - 1)

        k_sends = []
        v_sends = []
        for off in range(1, N_DEV):
            tgt = lax.rem(me + off, N_DEV)
            k_rdma = pltpu.make_async_remote_copy(
                src_ref=k_hbm.at[0, :, pl.ds(tgt * H_PER, H_PER), :],
                dst_ref=kg.at[me],
                send_sem=ksend.at[off - 1],
                recv_sem=krecv.at[off - 1],
                device_id=(tgt,),
                device_id_type=pl.DeviceIdType.MESH,
            )
            k_rdma.start()
            k_sends.append(k_rdma)
            v_rdma = pltpu.make_async_remote_copy(
                src_ref=v_hbm.at[0, :, pl.ds(tgt * H_PER, H_PER), :],
                dst_ref=vg.at[me],
                send_sem=vsend.at[off - 1],
                recv_sem=vrecv.at[off - 1],
                device_id=(tgt,),
                device_id_type=pl.DeviceIdType.MESH,
            )
            v_rdma.start()
            v_sends.append(v_rdma)

        k_loc = pltpu.make_async_copy(
            k_hbm.at[0, :, pl.ds(me * H_PER, H_PER), :],
            kg.at[me],
            krecv.at[N_DEV - 1],
        )
        k_loc.start()
        v_loc = pltpu.make_async_copy(
            v_hbm.at[0, :, pl.ds(me * H_PER, H_PER), :],
            vg.at[me],
            vrecv.at[N_DEV - 1],
        )
        v_loc.start()

        q_s[...] = lax.dot_general(
            x_ref[0], wq_ref[...],
            (((1,), (0,)), ((), ())),
            preferred_element_type=jnp.float32,
        ).astype(jnp.bfloat16)

        k_loc.wait()
        v_loc.wait()
        for off in range(1, N_DEV):
            pltpu.make_async_remote_copy(
                src_ref=kg.at[me], dst_ref=kg.at[me],
                send_sem=ksend.at[off - 1], recv_sem=krecv.at[off - 1],
                device_id=(me,), device_id_type=pl.DeviceIdType.MESH,
            ).wait_recv()
            pltpu.make_async_remote_copy(
                src_ref=vg.at[me], dst_ref=vg.at[me],
                send_sem=vsend.at[off - 1], recv_sem=vrecv.at[off - 1],
                device_id=(me,), device_id_type=pl.DeviceIdType.MESH,
            ).wait_recv()
        for r in k_sends:
            r.wait_send()
        for r in v_sends:
            r.wait_send()

        for h in range(H_PER):
            k_h = kg[:, :, h, :].reshape(SKV, DH)
            v_h = vg[:, :, h, :].reshape(SKV, DH)

            def qblk_body(qb, _, k_h=k_h, v_h=v_h, h=h):
                q_blk = q_s[pl.ds(qb * QBLK, QBLK), h * DH:(h + 1) * DH]
                s = lax.dot_general(
                    q_blk, k_h,
                    (((1,), (1,)), ((), ())),
                    preferred_element_type=jnp.float32,
                ) * SCALE
                qpos = qb * QBLK + lax.broadcasted_iota(
                    jnp.int32, (QBLK, SKV), 0)
                kpos = lax.broadcasted_iota(jnp.int32, (QBLK, SKV), 1)
                qb_id = qpos // 64
                kb_id = kpos // 64
                mask = (qb_id == kb_id) | (kb_id == 0) | (
                    lax.rem(qb_id + kb_id, 3) == 0)
                s = jnp.where(mask, s, -1e9)
                m = jnp.max(s, axis=1, keepdims=True)
                e = jnp.exp(s - m)
                den = jnp.sum(e, axis=1, keepdims=True)
                w = (e / den).astype(jnp.bfloat16)
                blk = lax.dot_general(
                    w, v_h,
                    (((1,), (0,)), ((), ())),
                    preferred_element_type=jnp.float32,
                )
                ctx_s[pl.ds(qb * QBLK, QBLK), h * DH:(h + 1) * DH] = (
                    blk.astype(jnp.bfloat16))
                return 0

            lax.fori_loop(0, SQ // QBLK, qblk_body, 0)

        partial = lax.dot_general(
            ctx_s[...], wo_ref[...],
            (((1,), (0,)), ((), ())),
            preferred_element_type=jnp.float32,
        )
        p_s[...] = partial.astype(jnp.bfloat16).reshape(SQ, H_PER, DH)

        for off in range(1, N_DEV):
            peer = lax.rem(me + off, N_DEV)
            pl.semaphore_signal(
                credit, inc=1,
                device_id=(peer,), device_id_type=pl.DeviceIdType.MESH,
            )
        pl.semaphore_wait(credit, N_DEV - 1)

        p_sends = []
        for off in range(1, N_DEV):
            tgt = lax.rem(me + off, N_DEV)
            rdma = pltpu.make_async_remote_copy(
                src_ref=p_s,
                dst_ref=vg.at[off - 1],
                send_sem=vsend.at[off - 1],
                recv_sem=vrecv.at[off - 1],
                device_id=(tgt,),
                device_id_type=pl.DeviceIdType.MESH,
            )
            rdma.start()
            p_sends.append(rdma)
        for off in range(1, N_DEV):
            pltpu.make_async_remote_copy(
                src_ref=p_s, dst_ref=vg.at[off - 1],
                send_sem=vsend.at[off - 1], recv_sem=vrecv.at[off - 1],
                device_id=(me,), device_id_type=pl.DeviceIdType.MESH,
            ).wait_recv()
        for r in p_sends:
            r.wait_send()

        acc = partial.reshape(SQ, H_PER, DH)
        for o in range(N_DEV - 1):
            acc = acc + vg[o].astype(jnp.float32)
        out_ref[0] = acc.reshape(SQ, N_DEV * H_PER * DH // 8)

    return pl.pallas_call(
        body,
        out_shape=jax.ShapeDtypeStruct((1, SQ, 1024), jnp.float32),
        in_specs=[
            pl.BlockSpec(memory_space=pltpu.VMEM),
            pl.BlockSpec(memory_space=pltpu.VMEM),
            pl.BlockSpec(memory_space=pltpu.ANY),
            pl.BlockSpec(memory_space=pltpu.ANY),
            pl.BlockSpec(memory_space=pltpu.VMEM),
        ],
        out_specs=pl.BlockSpec(memory_space=pltpu.VMEM),
        scratch_shapes=[
            pltpu.VMEM((N_DEV, KV_PER, H_PER, DH), jnp.bfloat16),
            pltpu.VMEM((N_DEV, KV_PER, H_PER, DH), jnp.bfloat16),
            pltpu.VMEM((SQ, H_PER * DH), jnp.bfloat16),
            pltpu.VMEM((SQ, H_PER * DH), jnp.bfloat16),
            pltpu.VMEM((SQ, H_PER, DH), jnp.bfloat16),
            pltpu.SemaphoreType.DMA((N_DEV - 1,)),
            pltpu.SemaphoreType.DMA((N_DEV,)),
            pltpu.SemaphoreType.DMA((N_DEV - 1,)),
            pltpu.SemaphoreType.DMA((N_DEV,)),
            pltpu.SemaphoreType.REGULAR,
        ],
        compiler_params=pltpu.CompilerParams(collective_id=0),
    )(xb, wqb, kb, vb, wob)


# baseline (device time: 895690 ns/iter reference)
import os

import jax
import jax.numpy as jnp
from jax import lax
from jax.experimental import pallas as pl
from jax.experimental.pallas import tpu as pltpu

N_DEV = 8
SQ = 1024
SKV = 8192
KV_PER = 1024
H_PER = 8
DH = 128
QBLK = 128
SCALE = 0.08838834764831843
NEG = -1e9


def kernel(x, Wq, K_ext, V_ext, Wo):
    xb = x.astype(jnp.bfloat16)
    wqb = Wq.astype(jnp.bfloat16)
    wob = Wo.astype(jnp.bfloat16)
    kb = K_ext.astype(jnp.bfloat16)[0].transpose(1, 0, 2)
    vb = V_ext.astype(jnp.bfloat16)[0].transpose(1, 0, 2)

    def body(x_ref, wq_ref, k_hbm, v_hbm, wo_ref, out_ref,
             kg, vg, q_s, ctx_s, p_s,
             ksend, krecv, vsend, vrecv, credit):
        me = lax.axis_index("i")

        barrier = pltpu.get_barrier_semaphore()
        for off in range(1, N_DEV):
            peer = lax.rem(me + off, N_DEV)
            pl.semaphore_signal(
                barrier, inc=1,
                device_id=(peer,), device_id_type=pl.DeviceIdType.MESH,
            )
        pl.semaphore_wait(barrier, N_DEV - 1)

        k_sends = []
        v_sends = []
        for off in range(1, N_DEV):
            tgt = lax.rem(me + off, N_DEV)
            k_rdma = pltpu.make_async_remote_copy(
                src_ref=k_hbm.at[pl.ds(tgt * H_PER, H_PER), :, :],
                dst_ref=kg.at[:, me],
                send_sem=ksend.at[off - 1],
                recv_sem=krecv.at[off - 1],
                device_id=(tgt,),
                device_id_type=pl.DeviceIdType.MESH,
            )
            k_rdma.start()
            k_sends.append(k_rdma)
            v_rdma = pltpu.make_async_remote_copy(
                src_ref=v_hbm.at[pl.ds(tgt * H_PER, H_PER), :, :],
                dst_ref=vg.at[:, me],
                send_sem=vsend.at[off - 1],
                recv_sem=vrecv.at[off - 1],
                device_id=(tgt,),
                device_id_type=pl.DeviceIdType.MESH,
            )
            v_rdma.start()
            v_sends.append(v_rdma)

        k_loc = pltpu.make_async_copy(
            k_hbm.at[pl.ds(me * H_PER, H_PER), :, :],
            kg.at[:, me],
            krecv.at[N_DEV - 1],
        )
        k_loc.start()
        v_loc = pltpu.make_async_copy(
            v_hbm.at[pl.ds(me * H_PER, H_PER), :, :],
            vg.at[:, me],
            vrecv.at[N_DEV - 1],
        )
        v_loc.start()

        for h in range(H_PER):
            q_s[h] = lax.dot_general(
                x_ref[0], wq_ref[:, h * DH:(h + 1) * DH],
                (((1,), (0,)), ((), ())),
                preferred_element_type=jnp.float32,
            ).astype(jnp.bfloat16)

        k_loc.wait()
        v_loc.wait()
        for off in range(1, N_DEV):
            pltpu.make_async_remote_copy(
                src_ref=kg.at[:, me], dst_ref=kg.at[:, me],
                send_sem=ksend.at[off - 1], recv_sem=krecv.at[off - 1],
                device_id=(me,), device_id_type=pl.DeviceIdType.MESH,
            ).wait_recv()
            pltpu.make_async_remote_copy(
                src_ref=vg.at[:, me], dst_ref=vg.at[:, me],
                send_sem=vsend.at[off - 1], recv_sem=vrecv.at[off - 1],
                device_id=(me,), device_id_type=pl.DeviceIdType.MESH,
            ).wait_recv()
        for r in k_sends:
            r.wait_send()
        for r in v_sends:
            r.wait_send()

        def head_body(h, _):
            def qblk_body(qb, __):
                q_blk = q_s[h, pl.ds(qb * QBLK, QBLK), :]

                def kv_body(kc, carry):
                    m, l, acc = carry
                    k_c = kg[h, kc]
                    v_c = vg[h, kc]
                    s = lax.dot_general(
                        q_blk, k_c,
                        (((1,), (1,)), ((), ())),
                        preferred_element_type=jnp.float32,
                    ) * SCALE
                    qpos = qb * QBLK + lax.broadcasted_iota(
                        jnp.int32, (QBLK, KV_PER), 0)
                    kpos = kc * KV_PER + lax.broadcasted_iota(
                        jnp.int32, (QBLK, KV_PER), 1)
                    qb_id = qpos // 64
                    kb_id = kpos // 64
                    mask = (qb_id == kb_id) | (kb_id == 0) | (
                        lax.rem(qb_id + kb_id, 3) == 0)
                    s = jnp.where(mask, s, NEG)
                    m_new = jnp.maximum(
                        m, jnp.max(s, axis=1, keepdims=True))
                    corr = jnp.exp(m - m_new)
                    e = jnp.exp(s - m_new)
                    l_new = l * corr + jnp.sum(e, axis=1, keepdims=True)
                    pv = lax.dot_general(
                        e.astype(jnp.bfloat16), v_c,
                        (((1,), (0,)), ((), ())),
                        preferred_element_type=jnp.float32,
                    )
                    acc_new = acc * corr + pv
                    return m_new, l_new, acc_new

                m0 = jnp.full((QBLK, 1), -3e38, jnp.float32)
                l0 = jnp.zeros((QBLK, 1), jnp.float32)
                a0 = jnp.zeros((QBLK, DH), jnp.float32)
                m, l, acc = lax.fori_loop(
                    0, N_DEV, kv_body, (m0, l0, a0))
                ctx_s[h, pl.ds(qb * QBLK, QBLK), :] = (
                    acc / l).astype(jnp.bfloat16)
                return __

            return lax.fori_loop(0, SQ // QBLK, qblk_body, _)

        lax.fori_loop(0, H_PER, head_body, 0)

        def oproj_body(h, a):
            return a + lax.dot_general(
                ctx_s[h], wo_ref[pl.ds(h * DH, DH), :],
                (((1,), (0,)), ((), ())),
                preferred_element_type=jnp.float32,
            )
        partial = lax.fori_loop(
            0, H_PER, oproj_body, jnp.zeros((SQ, H_PER * DH), jnp.float32))
        p_s[...] = partial.astype(jnp.bfloat16).reshape(N_DEV, KV_PER, DH)

        for off in range(1, N_DEV):
            peer = lax.rem(me + off, N_DEV)
            pl.semaphore_signal(
                credit, inc=1,
                device_id=(peer,), device_id_type=pl.DeviceIdType.MESH,
            )
        pl.semaphore_wait(credit, N_DEV - 1)

        p_sends = []
        for off in range(1, N_DEV):
            tgt = lax.rem(me + off, N_DEV)
            rdma = pltpu.make_async_remote_copy(
                src_ref=p_s,
                dst_ref=vg.at[off - 1],
                send_sem=vsend.at[off - 1],
                recv_sem=vrecv.at[off - 1],
                device_id=(tgt,),
                device_id_type=pl.DeviceIdType.MESH,
            )
            rdma.start()
            p_sends.append(rdma)
        for off in range(1, N_DEV):
            pltpu.make_async_remote_copy(
                src_ref=p_s, dst_ref=vg.at[off - 1],
                send_sem=vsend.at[off - 1], recv_sem=vrecv.at[off - 1],
                device_id=(me,), device_id_type=pl.DeviceIdType.MESH,
            ).wait_recv()
        for r in p_sends:
            r.wait_send()

        def sum_body(o, a):
            return a + vg[o].astype(jnp.float32)
        total = lax.fori_loop(
            0, N_DEV - 1, sum_body, partial.reshape(N_DEV, KV_PER, DH))
        out_ref[0] = total.reshape(SQ, H_PER * DH)

    return pl.pallas_call(
        body,
        out_shape=jax.ShapeDtypeStruct((1, SQ, 1024), jnp.float32),
        in_specs=[
            pl.BlockSpec(memory_space=pltpu.VMEM),
            pl.BlockSpec(memory_space=pltpu.VMEM),
            pl.BlockSpec(memory_space=pl.ANY),
            pl.BlockSpec(memory_space=pl.ANY),
            pl.BlockSpec(memory_space=pltpu.VMEM),
        ],
        out_specs=pl.BlockSpec(memory_space=pltpu.VMEM),
        scratch_shapes=[
            pltpu.VMEM((H_PER, N_DEV, KV_PER, DH), jnp.bfloat16),
            pltpu.VMEM((H_PER, N_DEV, KV_PER, DH), jnp.bfloat16),
            pltpu.VMEM((H_PER, SQ, DH), jnp.bfloat16),
            pltpu.VMEM((H_PER, SQ, DH), jnp.bfloat16),
            pltpu.VMEM((N_DEV, KV_PER, DH), jnp.bfloat16),
            pltpu.SemaphoreType.DMA((N_DEV - 1,)),
            pltpu.SemaphoreType.DMA((N_DEV,)),
            pltpu.SemaphoreType.DMA((N_DEV - 1,)),
            pltpu.SemaphoreType.DMA((N_DEV,)),
            pltpu.SemaphoreType.REGULAR,
        ],
        compiler_params=pltpu.CompilerParams(
            collective_id=0,
            vmem_limit_bytes=60 * 1024 * 1024,
        ),
        interpret=(
            pltpu.InterpretParams()
            if os.environ.get("KERNEL_INTERPRET") == "1"
            else False
        ),
    )(xb, wqb, kb, vb, wob)


# device time: 708719 ns/iter; 1.2638x vs baseline; 1.2638x over previous
import os

import jax
import jax.numpy as jnp
from jax import lax
from jax.experimental import pallas as pl
from jax.experimental.pallas import tpu as pltpu

N_DEV = 8
SQ = 1024
SKV = 8192
KV_PER = 1024
H_PER = 8
DH = 128
QBLK = 256
SCALE = 0.08838834764831843
NEG = -1e9


def kernel(x, Wq, K_ext, V_ext, Wo):
    xb = x.astype(jnp.bfloat16)
    wqb = Wq.astype(jnp.bfloat16)
    wob = Wo.astype(jnp.bfloat16)
    kb = K_ext.astype(jnp.bfloat16)[0].transpose(1, 0, 2)
    vb = V_ext.astype(jnp.bfloat16)[0].transpose(1, 0, 2)

    def body(x_ref, wq_ref, k_hbm, v_hbm, wo_ref, out_ref,
             kg, vg, q_s, ctx_s, p_s, bias_s,
             ksend, krecv, vsend, vrecv, credit):
        me = lax.axis_index("i")

        barrier = pltpu.get_barrier_semaphore()
        for off in range(1, N_DEV):
            peer = lax.rem(me + off, N_DEV)
            pl.semaphore_signal(
                barrier, inc=1,
                device_id=(peer,), device_id_type=pl.DeviceIdType.MESH,
            )
        pl.semaphore_wait(barrier, N_DEV - 1)

        k_sends = []
        v_sends = []
        for off in range(1, N_DEV):
            tgt = lax.rem(me + off, N_DEV)
            k_rdma = pltpu.make_async_remote_copy(
                src_ref=k_hbm.at[pl.ds(tgt * H_PER, H_PER), :, :],
                dst_ref=kg.at[:, me],
                send_sem=ksend.at[off - 1],
                recv_sem=krecv.at[off - 1],
                device_id=(tgt,),
                device_id_type=pl.DeviceIdType.MESH,
            )
            k_rdma.start()
            k_sends.append(k_rdma)
            v_rdma = pltpu.make_async_remote_copy(
                src_ref=v_hbm.at[pl.ds(tgt * H_PER, H_PER), :, :],
                dst_ref=vg.at[:, me],
                send_sem=vsend.at[off - 1],
                recv_sem=vrecv.at[off - 1],
                device_id=(tgt,),
                device_id_type=pl.DeviceIdType.MESH,
            )
            v_rdma.start()
            v_sends.append(v_rdma)

        k_loc = pltpu.make_async_copy(
            k_hbm.at[pl.ds(me * H_PER, H_PER), :, :],
            kg.at[:, me],
            krecv.at[N_DEV - 1],
        )
        k_loc.start()
        v_loc = pltpu.make_async_copy(
            v_hbm.at[pl.ds(me * H_PER, H_PER), :, :],
            vg.at[:, me],
            vrecv.at[N_DEV - 1],
        )
        v_loc.start()

        for h in range(H_PER):
            q_s[h] = lax.dot_general(
                x_ref[0], wq_ref[:, h * DH:(h + 1) * DH],
                (((1,), (0,)), ((), ())),
                preferred_element_type=jnp.float32,
            ).astype(jnp.bfloat16)

        k_loc.wait()
        v_loc.wait()
        for off in range(1, N_DEV):
            pltpu.make_async_remote_copy(
                src_ref=kg.at[:, me], dst_ref=kg.at[:, me],
                send_sem=ksend.at[off - 1], recv_sem=krecv.at[off - 1],
                device_id=(me,), device_id_type=pl.DeviceIdType.MESH,
            ).wait_recv()
            pltpu.make_async_remote_copy(
                src_ref=vg.at[:, me], dst_ref=vg.at[:, me],
                send_sem=vsend.at[off - 1], recv_sem=vrecv.at[off - 1],
                device_id=(me,), device_id_type=pl.DeviceIdType.MESH,
            ).wait_recv()
        for r in k_sends:
            r.wait_send()
        for r in v_sends:
            r.wait_send()

        def qblk_body(qb, _):
            def bias_body(kc, __):
                qpos = qb * QBLK + lax.broadcasted_iota(
                    jnp.int32, (QBLK, KV_PER), 0)
                kpos = kc * KV_PER + lax.broadcasted_iota(
                    jnp.int32, (QBLK, KV_PER), 1)
                qb_id = qpos // 64
                kb_id = kpos // 64
                mask = (qb_id == kb_id) | (kb_id == 0) | (
                    lax.rem(qb_id + kb_id, 3) == 0)
                bias_s[kc] = jnp.where(mask, 0.0, NEG).astype(jnp.bfloat16)
                return __

            lax.fori_loop(0, N_DEV, bias_body, 0)

            def head_body(h, __):
                q_blk = q_s[h, pl.ds(qb * QBLK, QBLK), :]

                def kv_body(kc, carry):
                    m, l, acc = carry
                    k_c = kg[h, kc]
                    v_c = vg[h, kc]
                    s = lax.dot_general(
                        q_blk, k_c,
                        (((1,), (1,)), ((), ())),
                        preferred_element_type=jnp.float32,
                    ) * SCALE + bias_s[kc]
                    m_new = jnp.maximum(
                        m, jnp.max(s, axis=1, keepdims=True))
                    corr = jnp.exp(m - m_new)
                    e = jnp.exp(s - m_new)
                    l_new = l * corr + jnp.sum(e, axis=1, keepdims=True)
                    pv = lax.dot_general(
                        e.astype(jnp.bfloat16), v_c,
                        (((1,), (0,)), ((), ())),
                        preferred_element_type=jnp.float32,
                    )
                    acc_new = acc * corr + pv
                    return m_new, l_new, acc_new

                m0 = jnp.full((QBLK, 1), -3e38, jnp.float32)
                l0 = jnp.zeros((QBLK, 1), jnp.float32)
                a0 = jnp.zeros((QBLK, DH), jnp.float32)
                m, l, acc = lax.fori_loop(
                    0, N_DEV, kv_body, (m0, l0, a0))
                ctx_s[h, pl.ds(qb * QBLK, QBLK), :] = (
                    acc / l).astype(jnp.bfloat16)
                return __

            return lax.fori_loop(0, H_PER, head_body, _)

        lax.fori_loop(0, SQ // QBLK, qblk_body, 0)

        def oproj_body(h, a):
            return a + lax.dot_general(
                ctx_s[h], wo_ref[pl.ds(h * DH, DH), :],
                (((1,), (0,)), ((), ())),
                preferred_element_type=jnp.float32,
            )
        partial = lax.fori_loop(
            0, H_PER, oproj_body, jnp.zeros((SQ, H_PER * DH), jnp.float32))
        p_s[...] = partial.astype(jnp.bfloat16).reshape(N_DEV, KV_PER, DH)

        for off in range(1, N_DEV):
            peer = lax.rem(me + off, N_DEV)
            pl.semaphore_signal(
                credit, inc=1,
                device_id=(peer,), device_id_type=pl.DeviceIdType.MESH,
            )
        pl.semaphore_wait(credit, N_DEV - 1)

        p_sends = []
        for off in range(1, N_DEV):
            tgt = lax.rem(me + off, N_DEV)
            rdma = pltpu.make_async_remote_copy(
                src_ref=p_s,
                dst_ref=vg.at[off - 1],
                send_sem=vsend.at[off - 1],
                recv_sem=vrecv.at[off - 1],
                device_id=(tgt,),
                device_id_type=pl.DeviceIdType.MESH,
            )
            rdma.start()
            p_sends.append(rdma)
        for off in range(1, N_DEV):
            pltpu.make_async_remote_copy(
                src_ref=p_s, dst_ref=vg.at[off - 1],
                send_sem=vsend.at[off - 1], recv_sem=vrecv.at[off - 1],
                device_id=(me,), device_id_type=pl.DeviceIdType.MESH,
            ).wait_recv()
        for r in p_sends:
            r.wait_send()

        def sum_body(o, a):
            return a + vg[o].astype(jnp.float32)
        total = lax.fori_loop(
            0, N_DEV - 1, sum_body, partial.reshape(N_DEV, KV_PER, DH))
        out_ref[0] = total.reshape(SQ, H_PER * DH)

    return pl.pallas_call(
        body,
        out_shape=jax.ShapeDtypeStruct((1, SQ, 1024), jnp.float32),
        in_specs=[
            pl.BlockSpec(memory_space=pltpu.VMEM),
            pl.BlockSpec(memory_space=pltpu.VMEM),
            pl.BlockSpec(memory_space=pl.ANY),
            pl.BlockSpec(memory_space=pl.ANY),
            pl.BlockSpec(memory_space=pltpu.VMEM),
        ],
        out_specs=pl.BlockSpec(memory_space=pltpu.VMEM),
        scratch_shapes=[
            pltpu.VMEM((H_PER, N_DEV, KV_PER, DH), jnp.bfloat16),
            pltpu.VMEM((H_PER, N_DEV, KV_PER, DH), jnp.bfloat16),
            pltpu.VMEM((H_PER, SQ, DH), jnp.bfloat16),
            pltpu.VMEM((H_PER, SQ, DH), jnp.bfloat16),
            pltpu.VMEM((N_DEV, KV_PER, DH), jnp.bfloat16),
            pltpu.VMEM((N_DEV, QBLK, KV_PER), jnp.bfloat16),
            pltpu.SemaphoreType.DMA((N_DEV - 1,)),
            pltpu.SemaphoreType.DMA((N_DEV,)),
            pltpu.SemaphoreType.DMA((N_DEV - 1,)),
            pltpu.SemaphoreType.DMA((N_DEV,)),
            pltpu.SemaphoreType.REGULAR,
        ],
        compiler_params=pltpu.CompilerParams(
            collective_id=0,
            vmem_limit_bytes=63 * 1024 * 1024,
        ),
        interpret=(
            pltpu.InterpretParams()
            if os.environ.get("KERNEL_INTERPRET") == "1"
            else False
        ),
    )(xb, wqb, kb, vb, wob)


# device time: 537796 ns/iter; 1.6655x vs baseline; 1.3178x over previous
import os

import jax
import jax.numpy as jnp
from jax import lax
from jax.experimental import pallas as pl
from jax.experimental.pallas import tpu as pltpu

N_DEV = 8
SQ = 1024
SKV = 8192
KV_PER = 1024
H_PER = 8
DH = 128
QBLK = 256
SCALE = 0.08838834764831843
NEG = -1e9


def kernel(x, Wq, K_ext, V_ext, Wo):
    xb = x.astype(jnp.bfloat16)
    wqb = Wq.astype(jnp.bfloat16)
    wob = Wo.astype(jnp.bfloat16)
    kb = K_ext.astype(jnp.bfloat16)[0].transpose(1, 0, 2)
    vb = V_ext.astype(jnp.bfloat16)[0].transpose(1, 0, 2)

    def body(x_ref, wq_ref, k_hbm, v_hbm, wo_ref, out_ref,
             kg, vg, q_s, p_s, acc_s, l_s,
             ksend, krecv, vsend, vrecv, credit):
        me = lax.axis_index("i")

        barrier = pltpu.get_barrier_semaphore()
        for off in range(1, N_DEV):
            peer = lax.rem(me + off, N_DEV)
            pl.semaphore_signal(
                barrier, inc=1,
                device_id=(peer,), device_id_type=pl.DeviceIdType.MESH,
            )
        pl.semaphore_wait(barrier, N_DEV - 1)

        k_sends = []
        v_sends = []
        for off in range(1, N_DEV):
            tgt = lax.rem(me + off, N_DEV)
            k_rdma = pltpu.make_async_remote_copy(
                src_ref=k_hbm.at[pl.ds(tgt * H_PER, H_PER), :, :],
                dst_ref=kg.at[:, me],
                send_sem=ksend.at[off - 1],
                recv_sem=krecv.at[off - 1],
                device_id=(tgt,),
                device_id_type=pl.DeviceIdType.MESH,
            )
            k_rdma.start()
            k_sends.append(k_rdma)
            v_rdma = pltpu.make_async_remote_copy(
                src_ref=v_hbm.at[pl.ds(tgt * H_PER, H_PER), :, :],
                dst_ref=vg.at[:, me],
                send_sem=vsend.at[off - 1],
                recv_sem=vrecv.at[off - 1],
                device_id=(tgt,),
                device_id_type=pl.DeviceIdType.MESH,
            )
            v_rdma.start()
            v_sends.append(v_rdma)

        k_loc = pltpu.make_async_copy(
            k_hbm.at[pl.ds(me * H_PER, H_PER), :, :],
            kg.at[:, me],
            krecv.at[N_DEV - 1],
        )
        k_loc.start()
        v_loc = pltpu.make_async_copy(
            v_hbm.at[pl.ds(me * H_PER, H_PER), :, :],
            vg.at[:, me],
            vrecv.at[N_DEV - 1],
        )
        v_loc.start()

        for h in range(H_PER):
            q_s[h] = lax.dot_general(
                x_ref[0], wq_ref[:, h * DH:(h + 1) * DH],
                (((1,), (0,)), ((), ())),
                preferred_element_type=jnp.float32,
            ).astype(jnp.bfloat16)

        acc_s[...] = jnp.zeros((H_PER, SQ, DH), jnp.float32)
        l_s[...] = jnp.zeros((H_PER, SQ // QBLK, 2, DH, 1), jnp.bfloat16)
        k_loc.wait()
        v_loc.wait()

        def chunk_body(o, _):
            kc = lax.rem(me - o + N_DEV, N_DEV)

            @pl.when(o > 0)
            def _wait_chunk():
                pltpu.make_async_remote_copy(
                    src_ref=kg.at[:, me], dst_ref=kg.at[:, me],
                    send_sem=ksend.at[o - 1], recv_sem=krecv.at[o - 1],
                    device_id=(me,), device_id_type=pl.DeviceIdType.MESH,
                ).wait_recv()
                pltpu.make_async_remote_copy(
                    src_ref=vg.at[:, me], dst_ref=vg.at[:, me],
                    send_sem=vsend.at[o - 1], recv_sem=vrecv.at[o - 1],
                    device_id=(me,), device_id_type=pl.DeviceIdType.MESH,
                ).wait_recv()

            def qblk_body(qb, __):
                qpos = qb * QBLK + lax.broadcasted_iota(
                    jnp.int32, (QBLK, KV_PER), 0)
                kpos = kc * KV_PER + lax.broadcasted_iota(
                    jnp.int32, (QBLK, KV_PER), 1)
                qb_id = qpos // 64
                kb_id = kpos // 64
                mask = (qb_id == kb_id) | (kb_id == 0) | (
                    lax.rem(qb_id + kb_id, 3) == 0)
                bias = jnp.where(mask, 0.0, NEG).astype(jnp.float32)

                def head_body(h, ___):
                    q_blk = q_s[h, pl.ds(qb * QBLK, QBLK), :]
                    s = lax.dot_general(
                        q_blk, kg[h, kc],
                        (((1,), (1,)), ((), ())),
                        preferred_element_type=jnp.float32,
                    ) * SCALE + bias
                    e = jnp.exp(s)
                    lsum = jnp.sum(
                        e.reshape(2, DH, KV_PER), axis=2, keepdims=True)
                    pv = lax.dot_general(
                        e.astype(jnp.bfloat16), vg[h, kc],
                        (((1,), (0,)), ((), ())),
                        preferred_element_type=jnp.float32,
                    )
                    acc_s[h, pl.ds(qb * QBLK, QBLK), :] = (
                        acc_s[h, pl.ds(qb * QBLK, QBLK), :] + pv)
                    l_s[h, qb] = (
                        l_s[h, qb].astype(jnp.float32) + lsum
                    ).astype(jnp.bfloat16)
                    return ___

                return lax.fori_loop(0, H_PER, head_body, __)

            lax.fori_loop(0, SQ // QBLK, qblk_body, 0)
            return _

        lax.fori_loop(0, N_DEV, chunk_body, 0)
        for r in k_sends:
            r.wait_send()
        for r in v_sends:
            r.wait_send()

        def oproj_body(h, a):
            ctx = (
                acc_s[h].reshape(SQ // QBLK, 2, DH, DH)
                / l_s[h].astype(jnp.float32)
            ).reshape(SQ, DH).astype(jnp.bfloat16)
            return a + lax.dot_general(
                ctx, wo_ref[pl.ds(h * DH, DH), :],
                (((1,), (0,)), ((), ())),
                preferred_element_type=jnp.float32,
            )
        partial = lax.fori_loop(
            0, H_PER, oproj_body, jnp.zeros((SQ, H_PER * DH), jnp.float32))
        p_s[...] = partial.astype(jnp.bfloat16).reshape(N_DEV, KV_PER, DH)

        for off in range(1, N_DEV):
            peer = lax.rem(me + off, N_DEV)
            pl.semaphore_signal(
                credit, inc=1,
                device_id=(peer,), device_id_type=pl.DeviceIdType.MESH,
            )
        pl.semaphore_wait(credit, N_DEV - 1)

        p_sends = []
        for off in range(1, N_DEV):
            tgt = lax.rem(me + off, N_DEV)
            rdma = pltpu.make_async_remote_copy(
                src_ref=p_s,
                dst_ref=vg.at[off - 1],
                send_sem=vsend.at[off - 1],
                recv_sem=vrecv.at[off - 1],
                device_id=(tgt,),
                device_id_type=pl.DeviceIdType.MESH,
            )
            rdma.start()
            p_sends.append(rdma)
        for off in range(1, N_DEV):
            pltpu.make_async_remote_copy(
                src_ref=p_s, dst_ref=vg.at[off - 1],
                send_sem=vsend.at[off - 1], recv_sem=vrecv.at[off - 1],
                device_id=(me,), device_id_type=pl.DeviceIdType.MESH,
            ).wait_recv()
        for r in p_sends:
            r.wait_send()

        def sum_body(o, a):
            return a + vg[o].astype(jnp.float32)
        total = lax.fori_loop(
            0, N_DEV - 1, sum_body, partial.reshape(N_DEV, KV_PER, DH))
        out_ref[0] = total.reshape(SQ, H_PER * DH)

    return pl.pallas_call(
        body,
        out_shape=jax.ShapeDtypeStruct((1, SQ, 1024), jnp.float32),
        in_specs=[
            pl.BlockSpec(memory_space=pltpu.VMEM),
            pl.BlockSpec(memory_space=pltpu.VMEM),
            pl.BlockSpec(memory_space=pl.ANY),
            pl.BlockSpec(memory_space=pl.ANY),
            pl.BlockSpec(memory_space=pltpu.VMEM),
        ],
        out_specs=pl.BlockSpec(memory_space=pltpu.VMEM),
        scratch_shapes=[
            pltpu.VMEM((H_PER, N_DEV, KV_PER, DH), jnp.bfloat16),
            pltpu.VMEM((H_PER, N_DEV, KV_PER, DH), jnp.bfloat16),
            pltpu.VMEM((H_PER, SQ, DH), jnp.bfloat16),
            pltpu.VMEM((N_DEV, KV_PER, DH), jnp.bfloat16),
            pltpu.VMEM((H_PER, SQ, DH), jnp.float32),
            pltpu.VMEM((H_PER, SQ // QBLK, 2, DH, 1), jnp.bfloat16),
            pltpu.SemaphoreType.DMA((N_DEV - 1,)),
            pltpu.SemaphoreType.DMA((N_DEV,)),
            pltpu.SemaphoreType.DMA((N_DEV - 1,)),
            pltpu.SemaphoreType.DMA((N_DEV,)),
            pltpu.SemaphoreType.REGULAR,
        ],
        compiler_params=pltpu.CompilerParams(
            collective_id=0,
            vmem_limit_bytes=63 * 1024 * 1024,
        ),
        interpret=(
            pltpu.InterpretParams()
            if os.environ.get("KERNEL_INTERPRET") == "1"
            else False
        ),
    )(xb, wqb, kb, vb, wob)
